# baseline (speedup 1.0000x reference)
import functools

import jax
import jax.numpy as jnp
from jax.experimental import pallas as pl
from jax.experimental.pallas import tpu as pltpu
from jax.experimental.pallas import tpu_sc as plsc

_B = 8
_D = 256
_K = 8192
_T = 576
_KC = 1024
_NKC = _K // _KC
_N = _B * _T

_NC = 2
_NS = 16
_NW = _NC * _NS
_BW = _N // _NW


def _vq_body(z_ref, cb_ref, oh_ref, idx_ref, ihist_ref, shist_ref,
             stats_ref, d_scr, ms_scr, idx_scr, mh_scr):
    kc = pl.program_id(1)

    @pl.when(kc == 0)
    def _pass1():
        z = z_ref[0]
        z2 = jnp.sum(z * z, axis=1, keepdims=True)
        minv = None
        idx = None
        s = None
        for c in range(_NKC):
            sl = slice(c * _KC, (c + 1) * _KC)
            cb = cb_ref[sl, :]
            c2 = jnp.sum(cb * cb, axis=1)
            zc = jax.lax.dot_general(z, cb, (((1,), (1,)), ((), ())),
                                     preferred_element_type=jnp.float32)
            d = (z2 + c2[None, :]) - 2.0 * zc
            mc = jnp.min(d, axis=1, keepdims=True)
            lane = jax.lax.broadcasted_iota(jnp.int32, (_T, _KC), 1)
            ic = jnp.min(jnp.where(d == mc, lane + c * _KC, _K), axis=1,
                         keepdims=True)
            if c == 0:
                minv, idx = mc, ic
            else:
                upd = mc < minv
                idx = jnp.where(upd, ic, idx)
                mnew = jnp.where(upd, mc, minv)
                s = s * jnp.exp(mnew - minv)
                minv = mnew
            e = jnp.exp(minv - d)
            d_scr[c] = e
            mh_scr[c] = minv
            se = jnp.sum(e, axis=1, keepdims=True)
            s = se if c == 0 else s + se

        ms_scr[:, 0:1] = minv
        ms_scr[:, 1:2] = 1.0 / s
        idx_flat = idx[:, 0]
        idx_scr[0, :] = idx_flat
        idx_ref[0, 0, :] = idx_flat

        sum_dmin = jnp.sum(minv)
        sum_logs = jnp.sum(jnp.log(s))
        lane128 = jax.lax.broadcasted_iota(jnp.int32, (1, 128), 1)
        stats_ref[0] = (jnp.where(lane128 == 0, sum_dmin, 0.0)
                        + jnp.where(lane128 == 1, sum_logs, 0.0))

    w = ms_scr[:, 1:2] * jnp.exp(ms_scr[:, 0:1] - mh_scr[kc])
    sh = jax.lax.dot_general(w, d_scr[kc], (((0,), (0,)), ((), ())),
                             preferred_element_type=jnp.float32)
    shist_ref[0, kc, 0, :] = sh[0]

    idx_row = idx_scr[0, :]
    kio = jax.lax.broadcasted_iota(jnp.int32, (_KC, _T), 0) + kc * _KC
    oh = (kio == idx_row[None, :]).astype(jnp.float32)
    oh_ref[0] = oh
    ones = jnp.ones((_T, 1), jnp.float32)
    ih = jax.lax.dot_general(oh, ones, (((1,), (0,)), ((), ())),
                             preferred_element_type=jnp.float32)
    ihist_ref[0, kc, 0, :] = ih[:, 0]


def _ent_body(ih_ref, out_ref):
    h = jnp.sum(ih_ref[...], axis=0)
    pr = h * (1.0 / _N)
    ent = jnp.sum(pr * jnp.log(pr + 1e-10))
    lane128 = jax.lax.broadcasted_iota(jnp.int32, (1, 128), 1)
    out_ref[...] = jnp.where(lane128 == 0, ent, 0.0)


def _sc_gather_body(cb_hbm, idx_hbm, out_hbm, idx_v, rows_v, sem):
    wid = jax.lax.axis_index("s") * _NC + jax.lax.axis_index("c")
    base = wid * _BW
    pltpu.sync_copy(idx_hbm.at[pl.ds(base, _BW)], idx_v)
    pltpu.async_copy(cb_hbm.at[idx_v], rows_v, sem).wait()
    pltpu.sync_copy(rows_v, out_hbm.at[pl.ds(base, _BW)])


def _sc_gather(codebook, idx_all):
    run = functools.partial(
        pl.kernel,
        out_type=jax.ShapeDtypeStruct((_N, _D), jnp.float32),
        mesh=plsc.VectorSubcoreMesh(core_axis_name="c", subcore_axis_name="s"),
        scratch_types=[
            pltpu.VMEM((_BW,), jnp.int32),
            pltpu.VMEM((_BW, _D), jnp.float32),
            pltpu.SemaphoreType.DMA,
        ],
    )(_sc_gather_body)
    return run(codebook, idx_all)


def kernel(z, codebook):
    z_bt = jnp.transpose(z, (0, 2, 3, 1)).reshape(_B, _T, _D)
    oh, idxo, ihist, shist, stats = pl.pallas_call(
        _vq_body,
        grid=(_B, _NKC),
        in_specs=[
            pl.BlockSpec((1, _T, _D), lambda b, kc: (b, 0, 0)),
            pl.BlockSpec((_K, _D), lambda b, kc: (0, 0)),
        ],
        out_specs=[
            pl.BlockSpec((1, _KC, _T), lambda b, kc: (b, kc, 0)),
            pl.BlockSpec((1, 1, _T), lambda b, kc: (b, 0, 0)),
            pl.BlockSpec((1, _NKC, 1, _KC), lambda b, kc: (b, 0, 0, 0)),
            pl.BlockSpec((1, _NKC, 1, _KC), lambda b, kc: (b, 0, 0, 0)),
            pl.BlockSpec((1, 1, 128), lambda b, kc: (b, 0, 0)),
        ],
        out_shape=[
            jax.ShapeDtypeStruct((_B, _K, _T), jnp.float32),
            jax.ShapeDtypeStruct((_B, 1, _T), jnp.int32),
            jax.ShapeDtypeStruct((_B, _NKC, 1, _KC), jnp.float32),
            jax.ShapeDtypeStruct((_B, _NKC, 1, _KC), jnp.float32),
            jax.ShapeDtypeStruct((_B, 1, 128), jnp.float32),
        ],
        scratch_shapes=[
            pltpu.VMEM((_NKC, _T, _KC), jnp.float32),
            pltpu.VMEM((_T, 128), jnp.float32),
            pltpu.VMEM((8, _T), jnp.int32),
            pltpu.VMEM((_NKC, _T, 1), jnp.float32),
        ],
        compiler_params=pltpu.CompilerParams(
            dimension_semantics=("arbitrary", "arbitrary")),
    )(z_bt, codebook)

    entout = pl.pallas_call(
        _ent_body,
        out_shape=jax.ShapeDtypeStruct((1, 128), jnp.float32),
    )(ihist)

    zq_rows = _sc_gather(codebook, idxo.reshape(_N))
    zq = jnp.transpose(zq_rows.reshape(_B, _T, _D), (0, 2, 1))

    dmin_tot = jnp.sum(stats[:, 0, 0])
    logs_tot = jnp.sum(stats[:, 0, 1])
    ent = entout[0, 0]
    closs = dmin_tot / (_N * _D)
    loss = 1.25 * closs
    sloss = logs_tot / _N
    perp = jnp.exp(-ent)
    z_q_ste = zq.reshape(_B, _D, 24, 24)
    onehot_out = oh.reshape(_B, _K, 24, 24)
    idx_out = idxo.reshape(_B, 1, 24, 24)
    index_histogram = ihist.reshape(_B, _K)
    softmax_histogram = shist.reshape(_B, _K)
    return (loss, z_q_ste, perp, onehot_out, idx_out, index_histogram,
            softmax_histogram, closs, closs, sloss)


# speedup vs baseline: 1.0324x; 1.0324x over previous
import functools

import jax
import jax.numpy as jnp
from jax.experimental import pallas as pl
from jax.experimental.pallas import tpu as pltpu
from jax.experimental.pallas import tpu_sc as plsc

_B = 8
_D = 256
_K = 8192
_T = 576
_KC = 1024
_NKC = _K // _KC
_N = _B * _T

_NC = 2
_NS = 16
_NW = _NC * _NS
_BW = _N // _NW


def _vq_body(z_ref, cb_ref, oh_ref, idx_ref, ihist_ref, shist_ref,
             stats_ref, d_scr, ms_scr, idx_scr, mh_scr):
    kc = pl.program_id(1)

    @pl.when(kc == 0)
    def _pass1():
        z = z_ref[0]
        z2 = jnp.sum(z * z, axis=1, keepdims=True)
        zz = z + z
        minv = None
        idx = None
        s = None
        for c in range(_NKC):
            sl = slice(c * _KC, (c + 1) * _KC)
            cb = cb_ref[sl, :]
            c2 = jnp.sum(cb * cb, axis=1)
            zc2 = jax.lax.dot_general(zz, cb, (((1,), (1,)), ((), ())),
                                      preferred_element_type=jnp.float32)
            d = (z2 + c2[None, :]) - zc2
            mc = jnp.min(d, axis=1, keepdims=True)
            lane = jax.lax.broadcasted_iota(jnp.int32, (_T, _KC), 1)
            ic = jnp.min(jnp.where(d == mc, lane + c * _KC, _K), axis=1,
                         keepdims=True)
            if c == 0:
                minv, idx = mc, ic
            else:
                upd = mc < minv
                idx = jnp.where(upd, ic, idx)
                mnew = jnp.where(upd, mc, minv)
                s = s * jnp.exp(mnew - minv)
                minv = mnew
            e = jnp.exp(minv - d)
            d_scr[c] = e
            mh_scr[c] = minv
            se = jnp.sum(e, axis=1, keepdims=True)
            s = se if c == 0 else s + se

        ms_scr[:, 0:1] = minv
        ms_scr[:, 1:2] = 1.0 / s
        idx_flat = idx[:, 0]
        idx_scr[0, :] = idx_flat
        idx_ref[0, 0, :] = idx_flat

        sum_dmin = jnp.sum(minv)
        sum_logs = jnp.sum(jnp.log(s))
        lane128 = jax.lax.broadcasted_iota(jnp.int32, (1, 128), 1)
        stats_ref[0] = (jnp.where(lane128 == 0, sum_dmin, 0.0)
                        + jnp.where(lane128 == 1, sum_logs, 0.0))

    w = ms_scr[:, 1:2] * jnp.exp(ms_scr[:, 0:1] - mh_scr[kc])
    sh = jax.lax.dot_general(w, d_scr[kc], (((0,), (0,)), ((), ())),
                             preferred_element_type=jnp.float32)
    shist_ref[0, kc, 0, :] = sh[0]

    idx_row = idx_scr[0, :]
    kio = jax.lax.broadcasted_iota(jnp.int32, (_KC, _T), 0) + kc * _KC
    oh = (kio == idx_row[None, :]).astype(jnp.float32)
    oh_ref[0] = oh
    ih = jnp.sum(oh, axis=1)
    ihist_ref[0, kc, 0, :] = ih


def _ent_body(ih_ref, out_ref):
    h = jnp.sum(ih_ref[...], axis=0)
    pr = h * (1.0 / _N)
    ent = jnp.sum(pr * jnp.log(pr + 1e-10))
    lane128 = jax.lax.broadcasted_iota(jnp.int32, (1, 128), 1)
    out_ref[...] = jnp.where(lane128 == 0, ent, 0.0)


def _sc_gather_body(cb_hbm, idx_hbm, out_hbm, idx_v, rows_v, sem):
    wid = jax.lax.axis_index("s") * _NC + jax.lax.axis_index("c")
    base = wid * _BW
    pltpu.sync_copy(idx_hbm.at[pl.ds(base, _BW)], idx_v)
    pltpu.async_copy(cb_hbm.at[idx_v], rows_v, sem).wait()
    pltpu.sync_copy(rows_v, out_hbm.at[pl.ds(base, _BW)])


def _sc_gather(codebook, idx_all):
    run = functools.partial(
        pl.kernel,
        out_type=jax.ShapeDtypeStruct((_N, _D), jnp.float32),
        mesh=plsc.VectorSubcoreMesh(core_axis_name="c", subcore_axis_name="s"),
        scratch_types=[
            pltpu.VMEM((_BW,), jnp.int32),
            pltpu.VMEM((_BW, _D), jnp.float32),
            pltpu.SemaphoreType.DMA,
        ],
    )(_sc_gather_body)
    return run(codebook, idx_all)


def kernel(z, codebook):
    z_bt = jnp.transpose(z, (0, 2, 3, 1)).reshape(_B, _T, _D)
    oh, idxo, ihist, shist, stats = pl.pallas_call(
        _vq_body,
        grid=(_B, _NKC),
        in_specs=[
            pl.BlockSpec((1, _T, _D), lambda b, kc: (b, 0, 0)),
            pl.BlockSpec((_K, _D), lambda b, kc: (0, 0)),
        ],
        out_specs=[
            pl.BlockSpec((1, _KC, _T), lambda b, kc: (b, kc, 0)),
            pl.BlockSpec((1, 1, _T), lambda b, kc: (b, 0, 0)),
            pl.BlockSpec((1, _NKC, 1, _KC), lambda b, kc: (b, 0, 0, 0)),
            pl.BlockSpec((1, _NKC, 1, _KC), lambda b, kc: (b, 0, 0, 0)),
            pl.BlockSpec((1, 1, 128), lambda b, kc: (b, 0, 0)),
        ],
        out_shape=[
            jax.ShapeDtypeStruct((_B, _K, _T), jnp.float32),
            jax.ShapeDtypeStruct((_B, 1, _T), jnp.int32),
            jax.ShapeDtypeStruct((_B, _NKC, 1, _KC), jnp.float32),
            jax.ShapeDtypeStruct((_B, _NKC, 1, _KC), jnp.float32),
            jax.ShapeDtypeStruct((_B, 1, 128), jnp.float32),
        ],
        scratch_shapes=[
            pltpu.VMEM((_NKC, _T, _KC), jnp.float32),
            pltpu.VMEM((_T, 128), jnp.float32),
            pltpu.VMEM((8, _T), jnp.int32),
            pltpu.VMEM((_NKC, _T, 1), jnp.float32),
        ],
        compiler_params=pltpu.CompilerParams(
            dimension_semantics=("arbitrary", "arbitrary")),
    )(z_bt, codebook)

    entout = pl.pallas_call(
        _ent_body,
        out_shape=jax.ShapeDtypeStruct((1, 128), jnp.float32),
    )(ihist)

    zq_rows = _sc_gather(codebook, idxo.reshape(_N))
    zq = jnp.transpose(zq_rows.reshape(_B, _T, _D), (0, 2, 1))

    dmin_tot = jnp.sum(stats[:, 0, 0])
    logs_tot = jnp.sum(stats[:, 0, 1])
    ent = entout[0, 0]
    closs = dmin_tot / (_N * _D)
    loss = 1.25 * closs
    sloss = logs_tot / _N
    perp = jnp.exp(-ent)
    z_q_ste = zq.reshape(_B, _D, 24, 24)
    onehot_out = oh.reshape(_B, _K, 24, 24)
    idx_out = idxo.reshape(_B, 1, 24, 24)
    index_histogram = ihist.reshape(_B, _K)
    softmax_histogram = shist.reshape(_B, _K)
    return (loss, z_q_ste, perp, onehot_out, idx_out, index_histogram,
            softmax_histogram, closs, closs, sloss)
